# baseline (speedup 1.0000x reference)
import functools

import jax
import jax.numpy as jnp
from jax.experimental import pallas as pl
from jax.experimental.pallas import tpu as pltpu

_K = 32


def _lrelu(t):
    return jnp.where(t >= 0, t, 0.1 * t)


def _det_body(x_ref, w1_ref, b1_ref, w2_ref, b2_ref, o_ref):
    h = _lrelu(jnp.dot(x_ref[...], w1_ref[...],
                       preferred_element_type=jnp.float32) + b1_ref[...])
    logit = jnp.dot(h, w2_ref[...], preferred_element_type=jnp.float32)
    o_ref[...] = jax.nn.sigmoid(logit + b2_ref[...])


def _detector(x2d, W1, b1, W2, b2, rb):
    M, H = x2d.shape
    return pl.pallas_call(
        _det_body,
        grid=(M // rb,),
        in_specs=[
            pl.BlockSpec((rb, H), lambda i: (i, 0)),
            pl.BlockSpec((H, H), lambda i: (0, 0)),
            pl.BlockSpec((1, H), lambda i: (0, 0)),
            pl.BlockSpec((H, 1), lambda i: (0, 0)),
            pl.BlockSpec((1, 1), lambda i: (0, 0)),
        ],
        out_specs=pl.BlockSpec((rb, 1), lambda i: (i, 0)),
        out_shape=jax.ShapeDtypeStruct((M, 1), jnp.float32),
    )(x2d, W1, b1.reshape(1, H), W2, b2.reshape(1, 1))


def _topk_body(p_ref, vals_ref, idx_ref, gidx_ref):
    B, S = p_ref.shape
    p0 = p_ref[...]
    col = jax.lax.broadcasted_iota(jnp.int32, (B, S), 1)
    kcol = jax.lax.broadcasted_iota(jnp.int32, (B, _K), 1)

    def body(j, carry):
        p, vals, idxs = carry
        m = jnp.max(p, axis=1, keepdims=True)
        cand = jnp.where(p == m, col, S)
        i = jnp.min(cand, axis=1, keepdims=True)
        vals = jnp.where(kcol == j, m, vals)
        idxs = jnp.where(kcol == j, i, idxs)
        p = jnp.where(col == i, -1.0, p)
        return p, vals, idxs

    _, vals, idxs = jax.lax.fori_loop(
        0, _K, body,
        (p0, jnp.zeros((B, _K), jnp.float32), jnp.zeros((B, _K), jnp.int32)))
    vals_ref[...] = vals
    idx_ref[...] = idxs
    row = jax.lax.broadcasted_iota(jnp.int32, (B, _K), 0)
    gidx_ref[...] = idxs + row * S


def _topk(probs):
    B, S = probs.shape
    return pl.pallas_call(
        _topk_body,
        out_shape=(
            jax.ShapeDtypeStruct((B, _K), jnp.float32),
            jax.ShapeDtypeStruct((B, _K), jnp.int32),
            jax.ShapeDtypeStruct((B, _K), jnp.int32),
        ),
    )(probs)


def _pn_body(gidx_ref, x_ref, w1_ref, b1_ref, w2_ref, b2_ref, idw_ref,
             idb_ref, gw_ref, gb_ref, tv_ref, ti_ref,
             tim_ref, ids_ref, g_ref, xg_ref, acc_ref, sem, *, nsteps):
    j = pl.program_id(0)
    R = xg_ref.shape[0]

    @pl.when(j == 0)
    def _():
        for r in range(R):
            pltpu.make_async_copy(x_ref.at[pl.ds(gidx_ref[r], 1)],
                                  xg_ref.at[pl.ds(r, 1)], sem).start()
        for r in range(R):
            pltpu.make_async_copy(x_ref.at[pl.ds(gidx_ref[r], 1)],
                                  xg_ref.at[pl.ds(r, 1)], sem).wait()

    f1 = _lrelu(jnp.dot(xg_ref[...], w1_ref[...],
                        preferred_element_type=jnp.float32) + b1_ref[...])
    part = jnp.dot(f1, w2_ref[...], preferred_element_type=jnp.float32)

    @pl.when(j == 0)
    def _():
        acc_ref[...] = part

    @pl.when(j > 0)
    def _():
        acc_ref[...] += part

    @pl.when(j == nsteps - 1)
    def _():
        N = idw_ref.shape[1]
        f2 = _lrelu(acc_ref[...] + b2_ref[...])
        logits = jnp.dot(f2, idw_ref[...],
                         preferred_element_type=jnp.float32) + idb_ref[...]
        m = jnp.max(logits, axis=1, keepdims=True)
        ncol = jax.lax.broadcasted_iota(jnp.int32, (R, N), 1)
        amax = jnp.min(jnp.where(logits == m, ncol, N), axis=1, keepdims=True)
        gl = jnp.sum(f2 * gw_ref[...], axis=1, keepdims=True) + gb_ref[...]
        gains = jax.nn.sigmoid(gl)
        mask = tv_ref[...] > 0.5
        tim_ref[...] = jnp.where(mask, ti_ref[...].astype(jnp.float32) * 0.01,
                                 0.0)
        ids_ref[...] = jnp.where(mask, amax, 0)
        g_ref[...] = jnp.where(mask, gains, 0.0)


def _param_net(x2d, gidx, W1, b1, W2, b2, idW, idb, gW, gb, tvals, tidx, cb):
    H = x2d.shape[1]
    R = gidx.shape[0]
    N = idW.shape[1]
    nsteps = H // cb
    body = functools.partial(_pn_body, nsteps=nsteps)
    grid_spec = pltpu.PrefetchScalarGridSpec(
        num_scalar_prefetch=1,
        grid=(nsteps,),
        in_specs=[
            pl.BlockSpec(memory_space=pl.ANY),
            pl.BlockSpec((H, cb), lambda j, gi: (0, j)),
            pl.BlockSpec((1, cb), lambda j, gi: (0, j)),
            pl.BlockSpec((cb, H), lambda j, gi: (j, 0)),
            pl.BlockSpec((1, H), lambda j, gi: (0, 0)),
            pl.BlockSpec((H, N), lambda j, gi: (0, 0)),
            pl.BlockSpec((1, N), lambda j, gi: (0, 0)),
            pl.BlockSpec((1, H), lambda j, gi: (0, 0)),
            pl.BlockSpec((1, 1), lambda j, gi: (0, 0)),
            pl.BlockSpec((R, 1), lambda j, gi: (0, 0)),
            pl.BlockSpec((R, 1), lambda j, gi: (0, 0)),
        ],
        out_specs=(
            pl.BlockSpec((R, 1), lambda j, gi: (0, 0)),
            pl.BlockSpec((R, 1), lambda j, gi: (0, 0)),
            pl.BlockSpec((R, 1), lambda j, gi: (0, 0)),
        ),
        scratch_shapes=[
            pltpu.VMEM((R, H), jnp.float32),
            pltpu.VMEM((R, H), jnp.float32),
            pltpu.SemaphoreType.DMA,
        ],
    )
    return pl.pallas_call(
        body,
        grid_spec=grid_spec,
        out_shape=(
            jax.ShapeDtypeStruct((R, 1), jnp.float32),
            jax.ShapeDtypeStruct((R, 1), jnp.int32),
            jax.ShapeDtypeStruct((R, 1), jnp.float32),
        ),
    )(gidx, x2d, W1, b1.reshape(1, H), W2, b2.reshape(1, H), idW,
      idb.reshape(1, N), gW.reshape(1, H), gb.reshape(1, 1), tvals, tidx)


def kernel(x, det_W1, det_b1, det_W2, det_b2, pn_W1, pn_b1, pn_W2, pn_b2,
           id_W, id_b, g_W, g_b):
    B, S, H = x.shape
    x2d = x.reshape(B * S, H)

    probs = _detector(x2d, det_W1, det_b1, det_W2, det_b2, rb=256)
    tvals, tidx, gidx = _topk(probs.reshape(B, S))
    tim, ids, gains = _param_net(
        x2d, gidx.reshape(-1), pn_W1, pn_b1, pn_W2, pn_b2, id_W, id_b,
        g_W, g_b, tvals.reshape(B * _K, 1), tidx.reshape(B * _K, 1), cb=512)
    return (tim.reshape(B, _K), ids.reshape(B, _K), gains.reshape(B, _K))
